# baseline (speedup 1.0000x reference)
import jax
import jax.numpy as jnp
import numpy as np
from jax.experimental import pallas as pl
from jax.experimental.pallas import tpu as pltpu

_N_LAYERS = 4
_BAND = _N_LAYERS + 3

_OFF_W0 = 0
_OFF_B0 = 128
_OFF_WS = 136
_OFF_BS = 648
_OFF_WE = 656
_OFF_BE = 784
_OFF_WFC = 792
_OFF_BFC = 856
_PACK_ROWS = 864


def _coeffs():
    es = [1.0] + [float(l + 3) for l in range(_N_LAYERS)] + [float(_N_LAYERS + 3)]
    out = []
    for e in es:
        dinv = np.float32(np.float32(e + 1.0) ** np.float32(-0.5))
        alpha = np.float32(np.float32(dinv * np.float32(e)) * dinv)
        beta = np.float32(dinv * dinv)
        out.append((alpha, beta))
    return out


_COEFFS = _coeffs()


def _body(scal_ref, emb_hbm, pk_hbm, out_ref, x_scr, pk_s, sem):
    copies = [
        pltpu.make_async_copy(
            emb_hbm.at[pl.ds(scal_ref[j], 1), :],
            x_scr.at[pl.ds(j, 1), :], sem)
        for j in range(_BAND)
    ] + [pltpu.make_async_copy(pk_hbm, pk_s, sem)]
    for c in copies:
        c.start()
    x_scr[pl.ds(_BAND, 1), :] = jnp.zeros((1, 128), jnp.float32)
    for c in copies:
        c.wait()

    def conv(x, w, b, k, relu):
        a, bt = _COEFFS[k]
        y = jnp.dot(x, w, preferred_element_type=jnp.float32)
        shifted = jnp.concatenate([jnp.zeros_like(y[:1]), y[:-1]], axis=0)
        y = a * shifted + bt * y + b
        return jnp.maximum(y, 0.0) if relu else y

    x = x_scr[...]
    x = conv(x, pk_s[_OFF_W0:_OFF_W0 + 128, :],
             pk_s[_OFF_B0:_OFF_B0 + 1, :], 0, True)
    for l in range(_N_LAYERS):
        x = conv(x, pk_s[_OFF_WS + 128 * l:_OFF_WS + 128 * (l + 1), :],
                 pk_s[_OFF_BS + l:_OFF_BS + l + 1, :], l + 1, True)
    x = conv(x, pk_s[_OFF_WE:_OFF_WE + 128, 0:64],
             pk_s[_OFF_BE:_OFF_BE + 1, 0:64], _N_LAYERS + 1, False)
    pre = jnp.dot(x, pk_s[_OFF_WFC:_OFF_WFC + 64, 0:50],
                  preferred_element_type=jnp.float32)
    pre = pre + pk_s[_OFF_BFC:_OFF_BFC + 1, 0:50]
    row = pre[_BAND - 1:_BAND, :]
    m = jnp.max(row, axis=1, keepdims=True)
    lse = m + jnp.log(jnp.sum(jnp.exp(row - m), axis=1, keepdims=True))
    lane = jax.lax.broadcasted_iota(jnp.int32, row.shape, 1)
    picked = jnp.sum(jnp.where(lane == scal_ref[_BAND], row, 0.0), axis=1,
                     keepdims=True)
    out_ref[...] = lse - picked


def kernel(batch_datas, batch_tags, emb_table, W_start, b_start, Ws, bs,
           W_end, b_end, W_fc, b_fc):
    n_vocab = emb_table.shape[0]
    tokens = jnp.clip(batch_datas[-1, -_BAND:], 0, n_vocab - 1)
    scal = jnp.concatenate([tokens, batch_tags])

    f32 = jnp.float32
    packed = jnp.concatenate([
        W_start,
        b_start.reshape(1, 128), jnp.zeros((7, 128), f32),
        Ws.reshape(_N_LAYERS * 128, 128),
        bs, jnp.zeros((4, 128), f32),
        jnp.pad(W_end, ((0, 0), (0, 64))),
        jnp.pad(b_end, (0, 64)).reshape(1, 128), jnp.zeros((7, 128), f32),
        jnp.pad(W_fc, ((0, 0), (0, 78))),
        jnp.pad(b_fc, (0, 78)).reshape(1, 128), jnp.zeros((7, 128), f32),
    ], axis=0)

    grid_spec = pltpu.PrefetchScalarGridSpec(
        num_scalar_prefetch=1,
        grid=(1,),
        in_specs=[
            pl.BlockSpec(memory_space=pl.ANY),
            pl.BlockSpec(memory_space=pl.ANY),
        ],
        out_specs=pl.BlockSpec((1, 1), lambda i, s: (0, 0)),
        scratch_shapes=[
            pltpu.VMEM((8, 128), jnp.float32),
            pltpu.VMEM((_PACK_ROWS, 128), jnp.float32),
            pltpu.SemaphoreType.DMA,
        ],
    )

    res = pl.pallas_call(
        _body,
        grid_spec=grid_spec,
        out_shape=jax.ShapeDtypeStruct((1, 1), jnp.float32),
    )(scal, emb_table, packed)
    return res[0, 0]


# speedup vs baseline: 2.1330x; 2.1330x over previous
import jax
import jax.numpy as jnp
import numpy as np
from jax.experimental import pallas as pl
from jax.experimental.pallas import tpu as pltpu

_N_LAYERS = 4
_BAND = _N_LAYERS + 3


def _coeffs():
    es = [1.0] + [float(l + 3) for l in range(_N_LAYERS)] + [float(_N_LAYERS + 3)]
    out = []
    for e in es:
        dinv = np.float32(np.float32(e + 1.0) ** np.float32(-0.5))
        alpha = np.float32(np.float32(dinv * np.float32(e)) * dinv)
        beta = np.float32(dinv * dinv)
        out.append((alpha, beta))
    return out


_COEFFS = _coeffs()


def _body(tag_ref, datas_hbm, emb_hbm, w0_hbm, b0_hbm, ws_hbm, bs_hbm,
          we_hbm, be_ref, wfc_hbm, bfc_ref, out_ref,
          x_scr, tok_v, w0_s, b0_s, ws_s, bs_s, we_s, wfc_s, sem_t, sem):
    nb, seq = datas_hbm.shape
    ncols = seq % 128
    tile0 = seq - ncols
    tok_copy = pltpu.make_async_copy(
        datas_hbm.at[:, pl.ds(tile0, ncols)], tok_v, sem_t)
    tok_copy.start()
    wcopies = [
        pltpu.make_async_copy(w0_hbm, w0_s, sem),
        pltpu.make_async_copy(b0_hbm, b0_s, sem),
        pltpu.make_async_copy(ws_hbm, ws_s, sem),
        pltpu.make_async_copy(bs_hbm, bs_s, sem),
        pltpu.make_async_copy(we_hbm, we_s, sem),
        pltpu.make_async_copy(wfc_hbm, wfc_s, sem),
    ]
    for c in wcopies:
        c.start()
    x_scr[pl.ds(_BAND, 1), :] = jnp.zeros((1, 128), jnp.float32)
    tok_copy.wait()
    gcopies = [
        pltpu.make_async_copy(
            emb_hbm.at[pl.ds(tok_v[nb - 1, ncols - _BAND + j], 1), :],
            x_scr.at[pl.ds(j, 1), :], sem)
        for j in range(_BAND)
    ]
    for c in gcopies:
        c.start()
    for c in wcopies + gcopies:
        c.wait()

    def conv(x, w, b, k, relu):
        a, bt = _COEFFS[k]
        y = jnp.dot(x, w, preferred_element_type=jnp.float32)
        shifted = jnp.concatenate([jnp.zeros_like(y[:1]), y[:-1]], axis=0)
        y = a * shifted + bt * y + b
        return jnp.maximum(y, 0.0) if relu else y

    x = x_scr[...]
    x = conv(x, w0_s[...], b0_s[...], 0, True)
    for l in range(_N_LAYERS):
        x = conv(x, ws_s[l], bs_s[l:l + 1, :], l + 1, True)
    x = conv(x, we_s[...], be_ref[...], _N_LAYERS + 1, False)
    pre = jnp.dot(x, wfc_s[...], preferred_element_type=jnp.float32)
    pre = pre + bfc_ref[...]
    row = pre[_BAND - 1:_BAND, :]
    m = jnp.max(row, axis=1, keepdims=True)
    lse = m + jnp.log(jnp.sum(jnp.exp(row - m), axis=1, keepdims=True))
    lane = jax.lax.broadcasted_iota(jnp.int32, row.shape, 1)
    picked = jnp.sum(jnp.where(lane == tag_ref[0], row, 0.0), axis=1,
                     keepdims=True)
    out_ref[...] = lse - picked


def kernel(batch_datas, batch_tags, emb_table, W_start, b_start, Ws, bs,
           W_end, b_end, W_fc, b_fc):
    grid_spec = pltpu.PrefetchScalarGridSpec(
        num_scalar_prefetch=1,
        grid=(1,),
        in_specs=[
            pl.BlockSpec(memory_space=pl.ANY),
            pl.BlockSpec(memory_space=pl.ANY),
            pl.BlockSpec(memory_space=pl.ANY),
            pl.BlockSpec(memory_space=pl.ANY),
            pl.BlockSpec(memory_space=pl.ANY),
            pl.BlockSpec(memory_space=pl.ANY),
            pl.BlockSpec(memory_space=pl.ANY),
            pl.BlockSpec((1, 64), lambda i, tag: (0, 0)),
            pl.BlockSpec(memory_space=pl.ANY),
            pl.BlockSpec((1, 50), lambda i, tag: (0, 0)),
        ],
        out_specs=pl.BlockSpec((1, 1), lambda i, tag: (0, 0)),
        scratch_shapes=[
            pltpu.VMEM((8, 128), jnp.float32),
            pltpu.VMEM((2, 80), jnp.int32),
            pltpu.VMEM((128, 128), jnp.float32),
            pltpu.VMEM((1, 128), jnp.float32),
            pltpu.VMEM((_N_LAYERS, 128, 128), jnp.float32),
            pltpu.VMEM((_N_LAYERS, 128), jnp.float32),
            pltpu.VMEM((128, 64), jnp.float32),
            pltpu.VMEM((64, 50), jnp.float32),
            pltpu.SemaphoreType.DMA,
            pltpu.SemaphoreType.DMA,
        ],
    )

    res = pl.pallas_call(
        _body,
        grid_spec=grid_spec,
        out_shape=jax.ShapeDtypeStruct((1, 1), jnp.float32),
    )(
        batch_tags, batch_datas, emb_table,
        W_start, b_start.reshape(1, 128), Ws, bs,
        W_end, b_end.reshape(1, 64), W_fc, b_fc.reshape(1, 50),
    )
    return res[0, 0]
